# baseline (speedup 1.0000x reference)
import functools

import jax
import jax.numpy as jnp
from jax import lax
from jax.experimental import pallas as pl
from jax.experimental.pallas import tpu as pltpu
from jax.experimental.pallas import tpu_sc as plsc

_N_WORKERS = 32
_CHUNK = 12800
_N_STATES = 1000000
_STAGE_SEG = 10000


def _gather_call(table, flat_idx, total):
    per_worker = total // _N_WORKERS
    n_chunks = per_worker // _CHUNK
    mesh = plsc.VectorSubcoreMesh(core_axis_name="c", subcore_axis_name="s")

    @functools.partial(
        pl.kernel,
        mesh=mesh,
        out_type=jax.ShapeDtypeStruct((total,), jnp.float32),
        scratch_types=[
            pltpu.VMEM((_CHUNK,), jnp.int32),
            pltpu.VMEM((_CHUNK,), jnp.int32),
            pltpu.VMEM((_CHUNK,), jnp.float32),
            pltpu.VMEM((_CHUNK,), jnp.float32),
            pltpu.VMEM((_STAGE_SEG,), jnp.float32),
            pltpu.VMEM_SHARED((_N_STATES,), jnp.float32),
            pltpu.SemaphoreType.DMA,
            pltpu.SemaphoreType.DMA,
            pltpu.SemaphoreType.DMA,
            pltpu.SemaphoreType.DMA,
            pltpu.SemaphoreType.DMA,
        ],
    )
    def k(table_hbm, idx_hbm, out_hbm, idx_v0, idx_v1, rows_v0, rows_v1,
          stage_v, table_sp, sem_i0, sem_i1, sem_g, sem_o0, sem_o1):
        idx_v = (idx_v0, idx_v1)
        rows_v = (rows_v0, rows_v1)
        sem_i = (sem_i0, sem_i1)
        sem_o = (sem_o0, sem_o1)
        sid = lax.axis_index("s")
        wid = sid * 2 + lax.axis_index("c")
        base = wid * per_worker

        n_pieces = _N_STATES // _STAGE_SEG
        n_rounds = -(-n_pieces // 16)
        for p in range(n_rounds):
            piece = p * 16 + sid

            @pl.when(piece < n_pieces)
            def _stage():
                seg = pl.ds(piece * _STAGE_SEG, _STAGE_SEG)
                pltpu.sync_copy(table_hbm.at[seg], stage_v)
                pltpu.sync_copy(stage_v, table_sp.at[seg])

        plsc.subcore_barrier()

        def idx_load(i):
            return pltpu.async_copy(
                idx_hbm.at[pl.ds(base + i * _CHUNK, _CHUNK)],
                idx_v[i % 2],
                sem_i[i % 2],
            )

        loads = {0: idx_load(0), 1: idx_load(1)}
        stores = {}
        for i in range(n_chunks):
            b = i % 2
            loads[i].wait()
            if i - 2 in stores:
                stores[i - 2].wait()
            pltpu.async_copy(table_sp.at[idx_v[b]], rows_v[b], sem_g).wait()
            if i + 2 < n_chunks:
                loads[i + 2] = idx_load(i + 2)
            stores[i] = pltpu.async_copy(
                rows_v[b],
                out_hbm.at[pl.ds(base + i * _CHUNK, _CHUNK)],
                sem_o[b],
            )
        stores[n_chunks - 2].wait()
        stores[n_chunks - 1].wait()

    return k(table, flat_idx)


@functools.partial(jax.jit, static_argnames=("rows", "hist"))
def _run(table, state, rows, hist):
    zi = lax.optimization_barrier(jnp.int32(0))
    zf = lax.optimization_barrier(jnp.float32(0))
    flat = state.reshape(rows * hist) + zi
    out = _gather_call(table, flat, rows * hist)
    return out.reshape(rows, hist) + zf


def kernel(rew_matrix, state):
    return _run(rew_matrix, state, state.shape[0], state.shape[1])


# speedup vs baseline: 1.1294x; 1.1294x over previous
import functools

import jax
import jax.numpy as jnp
from jax import lax
from jax.experimental import pallas as pl
from jax.experimental.pallas import tpu as pltpu
from jax.experimental.pallas import tpu_sc as plsc

_N_WORKERS = 32
_CHUNK = 12800
_N_STATES = 1000000
_STAGE_SEG = 10000


def _gather_call(table, flat_idx, total):
    per_worker = total // _N_WORKERS
    n_chunks = per_worker // _CHUNK
    mesh = plsc.VectorSubcoreMesh(core_axis_name="c", subcore_axis_name="s")

    @functools.partial(
        pl.kernel,
        mesh=mesh,
        out_type=jax.ShapeDtypeStruct((total,), jnp.float32),
        scratch_types=[
            pltpu.VMEM((_CHUNK,), jnp.int32),
            pltpu.VMEM((_CHUNK,), jnp.int32),
            pltpu.VMEM((_CHUNK,), jnp.float32),
            pltpu.VMEM((_CHUNK,), jnp.float32),
            pltpu.VMEM((_STAGE_SEG,), jnp.float32),
            pltpu.VMEM_SHARED((_N_STATES,), jnp.float32),
            pltpu.SemaphoreType.DMA,
            pltpu.SemaphoreType.DMA,
            pltpu.SemaphoreType.DMA,
            pltpu.SemaphoreType.DMA,
            pltpu.SemaphoreType.DMA,
        ],
    )
    def k(table_hbm, idx_hbm, out_hbm, idx_v0, idx_v1, rows_v0, rows_v1,
          stage_v, table_sp, sem_i0, sem_i1, sem_g, sem_o0, sem_o1):
        idx_v = (idx_v0, idx_v1)
        rows_v = (rows_v0, rows_v1)
        sem_i = (sem_i0, sem_i1)
        sem_o = (sem_o0, sem_o1)
        sid = lax.axis_index("s")
        wid = sid * 2 + lax.axis_index("c")
        base = wid * per_worker

        n_pieces = _N_STATES // _STAGE_SEG
        n_rounds = -(-n_pieces // 16)
        for p in range(n_rounds):
            piece = p * 16 + sid

            @pl.when(piece < n_pieces)
            def _stage():
                seg = pl.ds(piece * _STAGE_SEG, _STAGE_SEG)
                pltpu.sync_copy(table_hbm.at[seg], stage_v)
                pltpu.sync_copy(stage_v, table_sp.at[seg])

        plsc.subcore_barrier()

        def idx_load(i):
            return pltpu.async_copy(
                idx_hbm.at[pl.ds(base + i * _CHUNK, _CHUNK)],
                idx_v[i % 2],
                sem_i[i % 2],
            )

        loads = {0: idx_load(0), 1: idx_load(1)}
        stores = {}
        for i in range(n_chunks):
            b = i % 2
            loads[i].wait()
            if i - 2 in stores:
                stores[i - 2].wait()
            pltpu.async_copy(table_sp.at[idx_v[b]], rows_v[b], sem_g).wait()
            if i + 2 < n_chunks:
                loads[i + 2] = idx_load(i + 2)
            stores[i] = pltpu.async_copy(
                rows_v[b],
                out_hbm.at[pl.ds(base + i * _CHUNK, _CHUNK)],
                sem_o[b],
            )
        stores[n_chunks - 2].wait()
        stores[n_chunks - 1].wait()

    return k(table, flat_idx)


@functools.partial(jax.jit, static_argnames=("total",))
def _run(table, flat_idx, total):
    return _gather_call(table, flat_idx, total)


def kernel(rew_matrix, state):
    flat = state.reshape(-1)
    out = _run(rew_matrix, flat, flat.shape[0])
    return out.reshape(state.shape)
